# Initial kernel scaffold; baseline (speedup 1.0000x reference)
import jax
import jax.numpy as jnp
from jax.experimental import pallas as pl


def kernel(x, W_edge, b_edge, W1, b1, W2, b2, Wf, bf):
    raise NotImplementedError("write your pallas kernel here")


# speedup vs baseline: 13.4061x; 13.4061x over previous
import functools
import jax
import jax.numpy as jnp
from jax import lax
from jax.experimental import pallas as pl
from jax.experimental.pallas import tpu as pltpu
from jax.experimental.pallas import tpu_sc as plsc

B, N, K = 8, 2048, 32
RB = 256
C0 = 64
NEG = -3.0e38


def _knn_body(xt_ref, xtb_ref, wd_ref, wv_ref, be_ref, idx_ref, u_ref, v_ref,
              d_scr):
    b = pl.program_id(0)
    xt = xt_ref[0]
    xblk = xtb_ref[0]
    inner = lax.dot_general(xblk, xt, (((0,), (0,)), ((), ())),
                            preferred_element_type=jnp.float32)
    xx = jnp.sum(xt * xt, axis=0, keepdims=True)
    xxb = jnp.sum(xblk * xblk, axis=0, keepdims=True)
    d = 2.0 * inner - jnp.transpose(xxb, (1, 0)) - xx
    d_scr[...] = d

    wd = wd_ref[...]
    wv = wv_ref[...]
    u_ref[0] = lax.dot_general(xblk, wd, (((0,), (1,)), ((), ())),
                               preferred_element_type=jnp.float32)
    v_ref[0] = lax.dot_general(xblk, wv, (((0,), (1,)), ((), ())),
                               preferred_element_type=jnp.float32) + be_ref[...]

    cols = lax.broadcasted_iota(jnp.int32, (RB, N), 1)

    def body(k, _):
        dd = d_scr[...]
        mx = jnp.max(dd, axis=1, keepdims=True)
        eq = dd == mx
        am = jnp.min(jnp.where(eq, cols, jnp.int32(1 << 30)), axis=1)
        idx_ref[0, pl.ds(k, 1), :] = (am + b * N)[None, :]
        d_scr[...] = jnp.where(cols == am[:, None], NEG, dd)
        return 0

    lax.fori_loop(0, K, body, 0)


def _knn_call(xt, wd, wv, be):
    grid = (B, N // RB)
    return pl.pallas_call(
        _knn_body,
        grid=grid,
        in_specs=[
            pl.BlockSpec((1, 3, N), lambda b, j: (b, 0, 0)),
            pl.BlockSpec((1, 3, RB), lambda b, j: (b, 0, j)),
            pl.BlockSpec((C0, 3), lambda b, j: (0, 0)),
            pl.BlockSpec((C0, 3), lambda b, j: (0, 0)),
            pl.BlockSpec((1, C0), lambda b, j: (0, 0)),
        ],
        out_specs=[
            pl.BlockSpec((1, K, RB), lambda b, j: (b, 0, j)),
            pl.BlockSpec((1, RB, C0), lambda b, j: (b, j, 0)),
            pl.BlockSpec((1, RB, C0), lambda b, j: (b, j, 0)),
        ],
        out_shape=[
            jax.ShapeDtypeStruct((B, K, N), jnp.int32),
            jax.ShapeDtypeStruct((B, N, C0), jnp.float32),
            jax.ShapeDtypeStruct((B, N, C0), jnp.float32),
        ],
        scratch_shapes=[pltpu.VMEM((RB, N), jnp.float32)],
    )(xt, xt, wd, wv, be)


R = B * N
NW = 32
PW = R // NW
CP = 8
NG = PW // CP

@functools.lru_cache(maxsize=None)
def _gmax_sc(fuse_relu_add):
    mesh = plsc.VectorSubcoreMesh(core_axis_name="c", subcore_axis_name="s")

    def body(*refs):
        if fuse_relu_add:
            table_hbm, gidx_hbm, v_hbm, out_hbm = refs[:4]
            idx_v, rows_v, out_v, v_v, sem = refs[4:]
        else:
            table_hbm, gidx_hbm, out_hbm = refs[:3]
            idx_v, rows_v, out_v, sem = refs[3:]
        wid = lax.axis_index("s") * 2 + lax.axis_index("c")
        base = wid * PW
        pltpu.sync_copy(gidx_hbm.at[pl.ds(base * K, PW * K)], idx_v)

        def chunk(g, _):
            pltpu.async_copy(
                table_hbm.at[idx_v.at[pl.ds(g * (CP * K), CP * K)]],
                rows_v, sem).wait()
            if fuse_relu_add:
                pltpu.sync_copy(v_hbm.at[pl.ds(base + g * CP, CP)], v_v)
            for p in range(CP):
                for cv in range(C0 // 16):
                    sl = pl.ds(cv * 16, 16)
                    acc = rows_v[p * K, sl]
                    for t in range(1, K):
                        acc = jnp.maximum(acc, rows_v[p * K + t, sl])
                    if fuse_relu_add:
                        acc = jnp.maximum(acc + v_v[p, sl], 0.0)
                    out_v[p, sl] = acc
            pltpu.sync_copy(out_v, out_hbm.at[pl.ds(base + g * CP, CP)])
            return 0

        lax.fori_loop(0, NG, chunk, 0)

    scratch = [
        pltpu.VMEM((PW * K,), jnp.int32),
        pltpu.VMEM((CP * K, C0), jnp.float32),
        pltpu.VMEM((CP, C0), jnp.float32),
    ]
    if fuse_relu_add:
        scratch.append(pltpu.VMEM((CP, C0), jnp.float32))
    scratch.append(pltpu.SemaphoreType.DMA)

    return functools.partial(
        pl.kernel, mesh=mesh,
        out_type=jax.ShapeDtypeStruct((R, C0), jnp.float32),
        compiler_params=pltpu.CompilerParams(use_tc_tiling_on_sc=False),
        scratch_types=scratch)(body)


def _gmax_plain(table, gidx):
    return _gmax_sc(False)(table, gidx)


def _gmax_relu(table, gidx, v):
    return _gmax_sc(True)(table, gidx, v)


def _mm_relu_body(m_ref, w_ref, b_ref, o_ref):
    o_ref[...] = jnp.maximum(
        lax.dot_general(m_ref[...], w_ref[...], (((1,), (1,)), ((), ())),
                        preferred_element_type=jnp.float32) + b_ref[...], 0.0)


def _mm_relu(m, w, bvec):
    rows = 2048
    return pl.pallas_call(
        _mm_relu_body,
        grid=(R // rows,),
        in_specs=[
            pl.BlockSpec((rows, m.shape[1]), lambda i: (i, 0)),
            pl.BlockSpec(w.shape, lambda i: (0, 0)),
            pl.BlockSpec((1, w.shape[0]), lambda i: (0, 0)),
        ],
        out_specs=pl.BlockSpec((rows, w.shape[0]), lambda i: (i, 0)),
        out_shape=jax.ShapeDtypeStruct((R, w.shape[0]), jnp.float32),
    )(m, w, bvec)


def _final_body(h0_ref, h1_ref, m2_ref, w2_ref, b2_ref,
                wf0_ref, wf1_ref, wf2_ref, bf_ref, o_ref):
    h2 = jnp.maximum(
        lax.dot_general(m2_ref[...], w2_ref[...], (((1,), (1,)), ((), ())),
                        preferred_element_type=jnp.float32) + b2_ref[...], 0.0)
    s = lax.dot_general(h0_ref[...], wf0_ref[...], (((1,), (1,)), ((), ())),
                        preferred_element_type=jnp.float32)
    s += lax.dot_general(h1_ref[...], wf1_ref[...], (((1,), (1,)), ((), ())),
                         preferred_element_type=jnp.float32)
    s += lax.dot_general(h2, wf2_ref[...], (((1,), (1,)), ((), ())),
                         preferred_element_type=jnp.float32)
    o_ref[0] = jnp.max(s + bf_ref[...], axis=0, keepdims=True)


def _final_call(h0, h1, m2, w2, b2, wf0, wf1, wf2, bf):
    F = 256
    return pl.pallas_call(
        _final_body,
        grid=(B,),
        in_specs=[
            pl.BlockSpec((N, C0), lambda b: (b, 0)),
            pl.BlockSpec((N, C0), lambda b: (b, 0)),
            pl.BlockSpec((N, C0), lambda b: (b, 0)),
            pl.BlockSpec((128, C0), lambda b: (0, 0)),
            pl.BlockSpec((1, 128), lambda b: (0, 0)),
            pl.BlockSpec((F, C0), lambda b: (0, 0)),
            pl.BlockSpec((F, C0), lambda b: (0, 0)),
            pl.BlockSpec((F, 128), lambda b: (0, 0)),
            pl.BlockSpec((1, F), lambda b: (0, 0)),
        ],
        out_specs=pl.BlockSpec((1, 1, F), lambda b: (b, 0, 0)),
        out_shape=jax.ShapeDtypeStruct((B, 1, F), jnp.float32),
    )(h0, h1, m2, w2, b2, wf0, wf1, wf2, bf).reshape(B, F)


@jax.jit
def kernel(x, W_edge, b_edge, W1, b1, W2, b2, Wf, bf):
    xt = jnp.transpose(x, (0, 2, 1))
    wd = W_edge[:, :3]
    wv = W_edge[:, 3:] - wd

    idx, u, v = _knn_call(xt, wd, wv, b_edge[None, :])
    gidx = jnp.transpose(idx, (0, 2, 1)).reshape(-1)

    u = u.reshape(R, C0)
    v = v.reshape(R, C0)
    h0 = _gmax_relu(u, gidx, v)
    m1 = _gmax_plain(h0, gidx)
    h1 = _mm_relu(m1, W1, b1[None, :])
    m2 = _gmax_plain(h1, gidx)

    return _final_call(
        h0, h1, m2, W2, b2[None, :],
        Wf[:, :C0], Wf[:, C0:2 * C0], Wf[:, 2 * C0:], bf[None, :])
